# Initial kernel scaffold; baseline (speedup 1.0000x reference)
import jax
import jax.numpy as jnp
from jax.experimental import pallas as pl


def kernel(feats, edge_index, W_rel, b_rel, W_res, b_res, gamma, beta):
    raise NotImplementedError("write your pallas kernel here")


# speedup vs baseline: 3.0890x; 3.0890x over previous
import functools

import jax
import jax.numpy as jnp
from jax import lax
from jax.experimental import pallas as pl
from jax.experimental.pallas import tpu as pltpu
from jax.experimental.pallas import tpu_sc as plsc

N_NODES = 10000
D = 128
EPS = 1e-5

NC = 2
NS = 16
NW = NC * NS
C = 128
NCHUNK = 80
EPT = C * NCHUNK
E_PAD = NW * EPT
AGG_ROWS = 10240
DUMMY_DST = 10100
ZROWS_PER_TILE = AGG_ROWS // NS
OROWS_PER_TILE = N_NODES // NS


def _lin_body(x_ref, wt_ref, b_ref, o_ref):
    o_ref[...] = (
        jnp.dot(x_ref[...], wt_ref[...], preferred_element_type=jnp.float32)
        + b_ref[...]
    )


def _tc_linear(x, wt, b):
    nblk = 10
    rows = N_NODES // nblk
    return pl.pallas_call(
        _lin_body,
        grid=(nblk,),
        in_specs=[
            pl.BlockSpec((rows, D), lambda i: (i, 0)),
            pl.BlockSpec((D, D), lambda i: (0, 0)),
            pl.BlockSpec((1, D), lambda i: (0, 0)),
        ],
        out_specs=pl.BlockSpec((rows, D), lambda i: (i, 0)),
        out_shape=jax.ShapeDtypeStruct((N_NODES, D), jnp.float32),
    )(x, wt, b)


@functools.lru_cache(maxsize=1)
def _sc_scatter_build():
    mesh = plsc.VectorSubcoreMesh(core_axis_name="c", subcore_axis_name="s")

    @functools.partial(
        pl.kernel,
        mesh=mesh,
        out_type=jax.ShapeDtypeStruct((NC, AGG_ROWS, D), jnp.float32),
        scratch_types=[
            pltpu.VMEM((NCHUNK, C), jnp.int32),
            pltpu.VMEM((NCHUNK, C), jnp.int32),
            pltpu.VMEM((C, D), jnp.float32),
            pltpu.VMEM_SHARED((AGG_ROWS, D), jnp.float32),
            pltpu.SemaphoreType.DMA,
        ],
    )
    def sc_scatter(h_hbm, src_hbm, dst_hbm, zrows_hbm, out_hbm,
                   sidx, didx, rows, agg, sem):
        cid = lax.axis_index("c")
        sid = lax.axis_index("s")
        wid = cid * NS + sid

        pltpu.sync_copy(zrows_hbm, rows)
        for k in range(ZROWS_PER_TILE // C):
            pltpu.sync_copy(rows, agg.at[pl.ds(sid * ZROWS_PER_TILE + k * C, C)])

        pltpu.sync_copy(src_hbm.at[wid], sidx)
        pltpu.sync_copy(dst_hbm.at[wid], didx)
        plsc.subcore_barrier()

        def body(j, carry):
            pltpu.async_copy(h_hbm.at[sidx.at[j]], rows, sem).wait()
            pltpu.sync_copy(rows, agg.at[didx.at[j]], add=True)
            return carry

        lax.fori_loop(0, NCHUNK, body, 0)
        plsc.subcore_barrier()

        for k in range(ZROWS_PER_TILE // C):
            r0 = sid * ZROWS_PER_TILE + k * C
            pltpu.sync_copy(agg.at[pl.ds(r0, C)], rows)
            pltpu.sync_copy(rows, out_hbm.at[cid, pl.ds(r0, C)])

    return sc_scatter


def _comb_body(p_ref, x_ref, wt_ref, b_ref, t_ref, s_ref, q_ref):
    new = jnp.maximum(p_ref[0] + p_ref[1], 0.0)
    res = jnp.maximum(
        jnp.dot(x_ref[...], wt_ref[...], preferred_element_type=jnp.float32)
        + b_ref[...],
        0.0,
    )
    t = new + res
    t_ref[...] = t
    s_ref[...] = jnp.broadcast_to(jnp.sum(t, axis=0), (1, 8, D))
    q_ref[...] = jnp.broadcast_to(jnp.sum(t * t, axis=0), (1, 8, D))


def _tc_combine(p, x, wt, b):
    nblk = 10
    rows = N_NODES // nblk
    return pl.pallas_call(
        _comb_body,
        grid=(nblk,),
        in_specs=[
            pl.BlockSpec((NC, rows, D), lambda i: (0, i, 0)),
            pl.BlockSpec((rows, D), lambda i: (i, 0)),
            pl.BlockSpec((D, D), lambda i: (0, 0)),
            pl.BlockSpec((1, D), lambda i: (0, 0)),
        ],
        out_specs=[
            pl.BlockSpec((rows, D), lambda i: (i, 0)),
            pl.BlockSpec((1, 8, D), lambda i: (i, 0, 0)),
            pl.BlockSpec((1, 8, D), lambda i: (i, 0, 0)),
        ],
        out_shape=[
            jax.ShapeDtypeStruct((N_NODES, D), jnp.float32),
            jax.ShapeDtypeStruct((nblk, 8, D), jnp.float32),
            jax.ShapeDtypeStruct((nblk, 8, D), jnp.float32),
        ],
    )(p, x, wt, b)


def _norm_body(t_ref, s_ref, q_ref, g_ref, bt_ref, o_ref):
    n = float(N_NODES)
    mean = jnp.sum(s_ref[:, 0, :], axis=0, keepdims=True) / n
    var = jnp.sum(q_ref[:, 0, :], axis=0, keepdims=True) / n - mean * mean
    inv = lax.rsqrt(var + EPS)
    o_ref[...] = (t_ref[...] - mean) * (inv * g_ref[...]) + bt_ref[...]


def _tc_norm(t, s, q, gamma, beta):
    nblk = 10
    rows = N_NODES // nblk
    return pl.pallas_call(
        _norm_body,
        grid=(nblk,),
        in_specs=[
            pl.BlockSpec((rows, D), lambda i: (i, 0)),
            pl.BlockSpec((nblk, 8, D), lambda i: (0, 0, 0)),
            pl.BlockSpec((nblk, 8, D), lambda i: (0, 0, 0)),
            pl.BlockSpec((1, D), lambda i: (0, 0)),
            pl.BlockSpec((1, D), lambda i: (0, 0)),
        ],
        out_specs=pl.BlockSpec((rows, D), lambda i: (i, 0)),
        out_shape=jax.ShapeDtypeStruct((N_NODES, D), jnp.float32),
    )(t, s, q, gamma, beta)


def kernel(feats, edge_index, W_rel, b_rel, W_res, b_res, gamma, beta):
    src = edge_index[0].astype(jnp.int32)
    dst = edge_index[1].astype(jnp.int32)
    pad = E_PAD - src.shape[0]
    src = jnp.concatenate([src, jnp.zeros((pad,), jnp.int32)])
    dst = jnp.concatenate([dst, jnp.full((pad,), DUMMY_DST, jnp.int32)])
    src3 = src.reshape(NW, NCHUNK, C)
    dst3 = dst.reshape(NW, NCHUNK, C)
    zrows = jnp.zeros((C, D), jnp.float32)

    h = _tc_linear(feats, W_rel.T, b_rel.reshape(1, D))
    p = _sc_scatter_build()(h, src3, dst3, zrows)[:, :N_NODES, :]
    t, s, q = _tc_combine(p, feats, W_res.T, b_res.reshape(1, D))
    return _tc_norm(t, s, q, gamma.reshape(1, D), beta.reshape(1, D))


# speedup vs baseline: 3.7270x; 1.2065x over previous
import functools

import jax
import jax.numpy as jnp
from jax import lax
from jax.experimental import pallas as pl
from jax.experimental.pallas import tpu as pltpu
from jax.experimental.pallas import tpu_sc as plsc

N_NODES = 10000
D = 128
EPS = 1e-5

NC = 2
NS = 16
NW = NC * NS
C = 128
NCHUNK = 80
EPT = C * NCHUNK
E_PAD = NW * EPT
AGG_ROWS = 10240
DUMMY_DST = 10100
ZROWS_PER_TILE = AGG_ROWS // NS
OROWS_PER_TILE = N_NODES // NS


def _lin_body(x_ref, wt_ref, b_ref, o_ref):
    o_ref[...] = (
        jnp.dot(x_ref[...], wt_ref[...], preferred_element_type=jnp.float32)
        + b_ref[...]
    )


def _tc_linear(x, wt, b):
    nblk = 10
    rows = N_NODES // nblk
    return pl.pallas_call(
        _lin_body,
        grid=(nblk,),
        in_specs=[
            pl.BlockSpec((rows, D), lambda i: (i, 0)),
            pl.BlockSpec((D, D), lambda i: (0, 0)),
            pl.BlockSpec((1, D), lambda i: (0, 0)),
        ],
        out_specs=pl.BlockSpec((rows, D), lambda i: (i, 0)),
        out_shape=jax.ShapeDtypeStruct((N_NODES, D), jnp.float32),
    )(x, wt, b)


@functools.lru_cache(maxsize=1)
def _sc_scatter_build():
    mesh = plsc.VectorSubcoreMesh(core_axis_name="c", subcore_axis_name="s")

    @functools.partial(
        pl.kernel,
        mesh=mesh,
        out_type=jax.ShapeDtypeStruct((NC, AGG_ROWS, D), jnp.float32),
        scratch_types=[
            pltpu.VMEM((NCHUNK // 2, C), jnp.int32),
            pltpu.VMEM((NCHUNK // 2, C), jnp.int32),
            pltpu.VMEM((C, D), jnp.float32),
            pltpu.VMEM((C, D), jnp.float32),
            pltpu.VMEM_SHARED((AGG_ROWS, D), jnp.float32),
            pltpu.SemaphoreType.DMA,
            pltpu.SemaphoreType.DMA,
        ],
    )
    def sc_scatter(h_hbm, src_hbm, dst_hbm, zrows_hbm, out_hbm,
                   sidx, didx, rows0, rows1, agg, sem0, sem1):
        cid = lax.axis_index("c")
        sid = lax.axis_index("s")
        wid = cid * NS + sid

        pltpu.sync_copy(zrows_hbm, rows0)
        for k in range(ZROWS_PER_TILE // C):
            pltpu.sync_copy(rows0, agg.at[pl.ds(sid * ZROWS_PER_TILE + k * C, C)])

        plsc.subcore_barrier()

        def gat(k, buf, sem):
            return pltpu.make_async_copy(h_hbm.at[sidx.at[k]], buf, sem)

        half = NCHUNK // 2
        for hh in range(2):
            pltpu.sync_copy(src_hbm.at[wid, pl.ds(hh * half, half)], sidx)
            pltpu.sync_copy(dst_hbm.at[wid, pl.ds(hh * half, half)], didx)
            gat(0, rows0, sem0).start()

            def body(j, carry):
                a = 2 * j
                b = a + 1
                gat(b, rows1, sem1).start()
                gat(a, rows0, sem0).wait()
                pltpu.sync_copy(rows0, agg.at[didx.at[a]], add=True)

                @pl.when(j < half // 2 - 1)
                def _prefetch():
                    gat(a + 2, rows0, sem0).start()

                gat(b, rows1, sem1).wait()
                pltpu.sync_copy(rows1, agg.at[didx.at[b]], add=True)
                return carry

            lax.fori_loop(0, half // 2, body, 0)
        plsc.subcore_barrier()

        for k in range(ZROWS_PER_TILE // C):
            r0 = sid * ZROWS_PER_TILE + k * C
            pltpu.sync_copy(agg.at[pl.ds(r0, C)], rows0)
            pltpu.sync_copy(rows0, out_hbm.at[cid, pl.ds(r0, C)])

    return sc_scatter


def _comb_body(p_ref, x_ref, wt_ref, b_ref, t_ref, s_ref, q_ref):
    new = jnp.maximum(p_ref[0] + p_ref[1], 0.0)
    res = jnp.maximum(
        jnp.dot(x_ref[...], wt_ref[...], preferred_element_type=jnp.float32)
        + b_ref[...],
        0.0,
    )
    t = new + res
    t_ref[...] = t
    s_ref[...] = jnp.broadcast_to(jnp.sum(t, axis=0), (1, 8, D))
    q_ref[...] = jnp.broadcast_to(jnp.sum(t * t, axis=0), (1, 8, D))


def _tc_combine(p, x, wt, b):
    nblk = 10
    rows = N_NODES // nblk
    return pl.pallas_call(
        _comb_body,
        grid=(nblk,),
        in_specs=[
            pl.BlockSpec((NC, rows, D), lambda i: (0, i, 0)),
            pl.BlockSpec((rows, D), lambda i: (i, 0)),
            pl.BlockSpec((D, D), lambda i: (0, 0)),
            pl.BlockSpec((1, D), lambda i: (0, 0)),
        ],
        out_specs=[
            pl.BlockSpec((rows, D), lambda i: (i, 0)),
            pl.BlockSpec((1, 8, D), lambda i: (i, 0, 0)),
            pl.BlockSpec((1, 8, D), lambda i: (i, 0, 0)),
        ],
        out_shape=[
            jax.ShapeDtypeStruct((N_NODES, D), jnp.float32),
            jax.ShapeDtypeStruct((nblk, 8, D), jnp.float32),
            jax.ShapeDtypeStruct((nblk, 8, D), jnp.float32),
        ],
    )(p, x, wt, b)


def _norm_body(t_ref, s_ref, q_ref, g_ref, bt_ref, o_ref):
    n = float(N_NODES)
    mean = jnp.sum(s_ref[:, 0, :], axis=0, keepdims=True) / n
    var = jnp.sum(q_ref[:, 0, :], axis=0, keepdims=True) / n - mean * mean
    inv = lax.rsqrt(var + EPS)
    o_ref[...] = (t_ref[...] - mean) * (inv * g_ref[...]) + bt_ref[...]


def _tc_norm(t, s, q, gamma, beta):
    nblk = 10
    rows = N_NODES // nblk
    return pl.pallas_call(
        _norm_body,
        grid=(nblk,),
        in_specs=[
            pl.BlockSpec((rows, D), lambda i: (i, 0)),
            pl.BlockSpec((nblk, 8, D), lambda i: (0, 0, 0)),
            pl.BlockSpec((nblk, 8, D), lambda i: (0, 0, 0)),
            pl.BlockSpec((1, D), lambda i: (0, 0)),
            pl.BlockSpec((1, D), lambda i: (0, 0)),
        ],
        out_specs=pl.BlockSpec((rows, D), lambda i: (i, 0)),
        out_shape=jax.ShapeDtypeStruct((N_NODES, D), jnp.float32),
    )(t, s, q, gamma, beta)


def kernel(feats, edge_index, W_rel, b_rel, W_res, b_res, gamma, beta):
    src = edge_index[0].astype(jnp.int32)
    dst = edge_index[1].astype(jnp.int32)
    pad = E_PAD - src.shape[0]
    src = jnp.concatenate([src, jnp.zeros((pad,), jnp.int32)])
    dst = jnp.concatenate([dst, jnp.full((pad,), DUMMY_DST, jnp.int32)])
    src3 = src.reshape(NW, NCHUNK, C)
    dst3 = dst.reshape(NW, NCHUNK, C)
    zrows = jnp.zeros((C, D), jnp.float32)

    h = _tc_linear(feats, W_rel.T, b_rel.reshape(1, D))
    p = _sc_scatter_build()(h, src3, dst3, zrows)
    t, s, q = _tc_combine(p, feats, W_res.T, b_res.reshape(1, D))
    return _tc_norm(t, s, q, gamma.reshape(1, D), beta.reshape(1, D))
